# baseline (speedup 1.0000x reference)
import functools

import jax
import jax.numpy as jnp
from jax import lax
from jax.experimental import pallas as pl
from jax.experimental.pallas import tpu as pltpu
from jax.experimental.pallas import tpu_sc as plsc

_NUM_CLASSES = 19
_P = 96 * 96
_C = 128
_NC = 2
_NS = 16
_NW = _NC * _NS
_BPW = _P // _NW
_GRP = _BPW // 16


def _hist_body(labt_hbm, cnt_hbm, lab_v, cnt_v):
    wid = lax.axis_index("s") * _NC + lax.axis_index("c")
    pltpu.sync_copy(labt_hbm.at[wid], lab_v)

    def group(g, carry):
        off = g * 16
        labs = [lab_v[l, pl.ds(off, 16)] for l in range(16)]
        for c in range(_NUM_CLASSES):
            acc = jnp.zeros((16,), jnp.float32)
            for l in range(16):
                acc = acc + jnp.where(labs[l] == c, 1.0, 0.0)
            cnt_v[c, pl.ds(off, 16)] = acc
        return carry

    lax.fori_loop(0, _GRP, group, 0)
    pltpu.sync_copy(cnt_v, cnt_hbm.at[wid])


@functools.cache
def _build_hist_sc():
    return pl.kernel(
        _hist_body,
        mesh=plsc.VectorSubcoreMesh(core_axis_name="c", subcore_axis_name="s",
                                    num_cores=_NC, num_subcores=_NS),
        out_type=jax.ShapeDtypeStruct((_NW, _NUM_CLASSES, _BPW), jnp.float32),
        scratch_types=[
            pltpu.VMEM((16, _BPW), jnp.int32),
            pltpu.VMEM((_NUM_CLASSES, _BPW), jnp.float32),
        ],
    )


def _loss_kernel(x_ref, cnt_ref, out_ref):
    x = x_ref[:]
    cnt = cnt_ref[:]

    dn = (((1,), (1,)), ((), ()))
    s1 = jax.lax.dot_general(x, cnt, dn, precision=jax.lax.Precision.HIGHEST,
                             preferred_element_type=jnp.float32)
    s2 = jax.lax.dot_general(x * x, cnt, dn,
                             precision=jax.lax.Precision.HIGHEST,
                             preferred_element_type=jnp.float32)

    kvec = jax.lax.broadcasted_iota(jnp.int32, (1, 7), 1).astype(jnp.float32) - 3.0
    tw = jnp.exp(-0.5 * kvec * kvec)
    target = tw / jnp.sum(tw)
    loss_acc = jnp.float32(0.0)
    act_acc = jnp.float32(0.0)
    for c in range(_NUM_CLASSES):
        cp = cnt[c:c + 1, :]
        n_c = jnp.sum(cp)
        nsafe = jnp.maximum(n_c, 1.0)
        mu = s1[:, c:c + 1] / nsafe
        e2 = s2[:, c:c + 1] / nsafe
        var = e2 - mu * mu * (2.0 - n_c / nsafe) + 1e-10
        inv_std = jax.lax.rsqrt(var)
        z = (mu - x) * inv_std
        us = []
        for k in range(-3, 4):
            zk = z + jnp.float32(k)
            e = jnp.exp(-12.5 * zk * zk)
            us.append(jax.lax.dot_general(
                e, cp, dn, precision=jax.lax.Precision.DEFAULT,
                preferred_element_type=jnp.float32))
        u = jnp.concatenate(us, axis=1)
        ssum = jnp.sum(u, axis=1, keepdims=True)
        hist = u / ssum
        d = jnp.abs(hist - target)
        sl = jnp.where(d < 1.0, 0.5 * d * d, d - 0.5)
        lc = jnp.sum(sl) * jnp.float32(1.0 / (_C * 7))
        active = n_c >= 1000.0
        loss_acc = loss_acc + jnp.where(active, lc, 0.0)
        act_acc = act_acc + jnp.where(active, 1.0, 0.0)

    out_ref[0, 0] = loss_acc / act_acc


def kernel(feature, label):
    x = feature[0].reshape(_C, _P)
    labt = (label[0, 0].astype(jnp.int32)
            .reshape(96, 4, 96, 4).transpose(1, 3, 0, 2).reshape(16, _P))
    labt32 = labt.reshape(16, _NW, _BPW).transpose(1, 0, 2)
    cnt32 = _build_hist_sc()(labt32)
    cnt = cnt32.transpose(1, 0, 2).reshape(_NUM_CLASSES, _P)
    out = pl.pallas_call(
        _loss_kernel,
        out_shape=jax.ShapeDtypeStruct((1, 1), jnp.float32),
        out_specs=pl.BlockSpec(memory_space=pltpu.SMEM),
    )(x, cnt)
    return out[0, 0]


# speedup vs baseline: 1.0144x; 1.0144x over previous
import functools

import jax
import jax.numpy as jnp
from jax import lax
from jax.experimental import pallas as pl
from jax.experimental.pallas import tpu as pltpu
from jax.experimental.pallas import tpu_sc as plsc

_NUM_CLASSES = 19
_P = 96 * 96
_C = 128
_NC = 2
_NS = 16
_NW = _NC * _NS
_CHUNK = 128
_NCHUNK = _P // _CHUNK
_TPW = (_NCHUNK + _NW - 1) // _NW


def _hist_body(labt_hbm, cnt_hbm, lab_v, cnt_v):
    wid = lax.axis_index("s") * _NC + lax.axis_index("c")
    for t in range(_TPW):
        cid = wid + _NW * t

        @pl.when(cid < _NCHUNK)
        def _():
            base = pl.multiple_of(cid * _CHUNK, _CHUNK)
            pltpu.sync_copy(labt_hbm.at[:, pl.ds(base, _CHUNK)], lab_v)

            def group(g, carry):
                off = g * 16
                labs = [lab_v[l, pl.ds(off, 16)] for l in range(16)]
                for c in range(_NUM_CLASSES):
                    acc = jnp.zeros((16,), jnp.float32)
                    for l in range(16):
                        acc = acc + jnp.where(labs[l] == c, 1.0, 0.0)
                    cnt_v[c, pl.ds(off, 16)] = acc
                return carry

            lax.fori_loop(0, _CHUNK // 16, group, 0)
            pltpu.sync_copy(cnt_v, cnt_hbm.at[:, pl.ds(base, _CHUNK)])


@functools.cache
def _build_hist_sc():
    return pl.kernel(
        _hist_body,
        mesh=plsc.VectorSubcoreMesh(core_axis_name="c", subcore_axis_name="s",
                                    num_cores=_NC, num_subcores=_NS),
        out_type=jax.ShapeDtypeStruct((_NUM_CLASSES, _P), jnp.float32),
        scratch_types=[
            pltpu.VMEM((16, _CHUNK), jnp.int32),
            pltpu.VMEM((_NUM_CLASSES, _CHUNK), jnp.float32),
        ],
    )


def _loss_kernel(x_ref, cnt_ref, out_ref):
    x = x_ref[:]
    cnt = cnt_ref[:]

    dn = (((1,), (1,)), ((), ()))
    s1 = jax.lax.dot_general(x, cnt, dn, precision=jax.lax.Precision.HIGHEST,
                             preferred_element_type=jnp.float32)
    s2 = jax.lax.dot_general(x * x, cnt, dn,
                             precision=jax.lax.Precision.HIGHEST,
                             preferred_element_type=jnp.float32)

    kvec = jax.lax.broadcasted_iota(jnp.int32, (1, 7), 1).astype(jnp.float32) - 3.0
    tw = jnp.exp(-0.5 * kvec * kvec)
    target = tw / jnp.sum(tw)

    em45 = jnp.float32(2.8625186e-20)
    em70 = jnp.float32(3.9754497e-31)
    e5 = jnp.float32(148.41316)
    scales = (em70, em45, em45, e5, em45, em45, em70)

    loss_acc = jnp.float32(0.0)
    act_acc = jnp.float32(0.0)
    for c in range(_NUM_CLASSES):
        cp = cnt[c:c + 1, :]
        n_c = jnp.sum(cp)
        nsafe = jnp.maximum(n_c, 1.0)
        mu = s1[:, c:c + 1] / nsafe
        e2 = s2[:, c:c + 1] / nsafe
        var = e2 - mu * mu * (2.0 - n_c / nsafe) + 1e-10
        inv_std = jax.lax.rsqrt(var)
        z = (mu - x) * inv_std
        zc = jnp.clip(z, -4.1, 4.1)
        q = -12.5 * zc * zc
        m = 25.0 * zc
        v = q + 32.5
        t1 = jnp.exp(v - m)
        w1 = jnp.exp(v + m)
        bp = jnp.exp(-37.5 - m)
        bn = jnp.exp(m - 37.5)
        t2 = t1 * bp
        t3 = t2 * bp
        w2 = w1 * bn
        w3 = w2 * bn
        e0 = t1 * bn
        us = []
        for arr, sc in zip((w3, w2, w1, e0, t1, t2, t3), scales):
            uk = jax.lax.dot_general(
                arr, cp, dn, precision=jax.lax.Precision.DEFAULT,
                preferred_element_type=jnp.float32)
            us.append(uk * sc)
        u = jnp.concatenate(us, axis=1)
        ssum = jnp.sum(u, axis=1, keepdims=True)
        hist = u / ssum
        d = jnp.abs(hist - target)
        sl = jnp.where(d < 1.0, 0.5 * d * d, d - 0.5)
        lc = jnp.sum(sl) * jnp.float32(1.0 / (_C * 7))
        active = n_c >= 1000.0
        loss_acc = loss_acc + jnp.where(active, lc, 0.0)
        act_acc = act_acc + jnp.where(active, 1.0, 0.0)

    out_ref[0, 0] = loss_acc / act_acc


def kernel(feature, label):
    x = feature[0].reshape(_C, _P)
    labt = (label[0, 0].astype(jnp.int32)
            .reshape(96, 4, 96, 4).transpose(1, 3, 0, 2).reshape(16, _P))
    cnt = _build_hist_sc()(labt)
    out = pl.pallas_call(
        _loss_kernel,
        out_shape=jax.ShapeDtypeStruct((1, 1), jnp.float32),
        out_specs=pl.BlockSpec(memory_space=pltpu.SMEM),
    )(x, cnt)
    return out[0, 0]


# speedup vs baseline: 1.1235x; 1.1075x over previous
import functools

import jax
import jax.numpy as jnp
from jax import lax
from jax.experimental import pallas as pl
from jax.experimental.pallas import tpu as pltpu
from jax.experimental.pallas import tpu_sc as plsc

_NUM_CLASSES = 19
_P = 96 * 96
_C = 128
_NC = 2
_NS = 16
_NW = _NC * _NS
_CHUNK = 128
_NCHUNK = _P // _CHUNK
_TPW = (_NCHUNK + _NW - 1) // _NW


def _hist_body(labt_hbm, cnt_hbm, lab_v, cnt_v):
    wid = lax.axis_index("s") * _NC + lax.axis_index("c")
    for t in range(_TPW):
        cid = wid + _NW * t

        @pl.when(cid < _NCHUNK)
        def _():
            base = pl.multiple_of(cid * _CHUNK, _CHUNK)
            pltpu.sync_copy(labt_hbm.at[:, pl.ds(base, _CHUNK)], lab_v)

            def group(g, carry):
                off = g * 16
                labs = [lab_v[l, pl.ds(off, 16)] for l in range(16)]
                for c in range(_NUM_CLASSES):
                    acc = jnp.zeros((16,), jnp.float32)
                    for l in range(16):
                        acc = acc + jnp.where(labs[l] == c, 1.0, 0.0)
                    cnt_v[c, pl.ds(off, 16)] = acc
                return carry

            lax.fori_loop(0, _CHUNK // 16, group, 0)
            pltpu.sync_copy(cnt_v, cnt_hbm.at[:, pl.ds(base, _CHUNK)])


@functools.cache
def _build_hist_sc():
    return pl.kernel(
        _hist_body,
        mesh=plsc.VectorSubcoreMesh(core_axis_name="c", subcore_axis_name="s",
                                    num_cores=_NC, num_subcores=_NS),
        out_type=jax.ShapeDtypeStruct((_NUM_CLASSES, _P), jnp.float32),
        scratch_types=[
            pltpu.VMEM((16, _CHUNK), jnp.int32),
            pltpu.VMEM((_NUM_CLASSES, _CHUNK), jnp.float32),
        ],
    )


def _loss_kernel(x_ref, cnt_ref, out_ref):
    x = x_ref[:]
    cnt = cnt_ref[:]

    dn = (((1,), (1,)), ((), ()))
    s1 = jax.lax.dot_general(x, cnt, dn, precision=jax.lax.Precision.HIGHEST,
                             preferred_element_type=jnp.float32)
    s2 = jax.lax.dot_general(x * x, cnt, dn,
                             precision=jax.lax.Precision.HIGHEST,
                             preferred_element_type=jnp.float32)

    kvec = jax.lax.broadcasted_iota(jnp.int32, (1, 7), 1).astype(jnp.float32) - 3.0
    tw = jnp.exp(-0.5 * kvec * kvec)
    target = tw / jnp.sum(tw)

    em45 = jnp.float32(2.8625186e-20)
    em70 = jnp.float32(3.9754497e-31)
    e5 = jnp.float32(148.41316)
    scales = (em70, em45, em45, e5, em45, em45, em70)

    loss_acc = jnp.float32(0.0)
    act_acc = jnp.float32(0.0)
    for c in range(_NUM_CLASSES):
        cp = cnt[c:c + 1, :]
        n_c = jnp.sum(cp)
        nsafe = jnp.maximum(n_c, 1.0)
        mu = s1[:, c:c + 1] / nsafe
        e2 = s2[:, c:c + 1] / nsafe
        var = e2 - mu * mu * (2.0 - n_c / nsafe) + 1e-10
        inv_std = jax.lax.rsqrt(var)
        z = (mu - x) * inv_std
        zc = jnp.clip(z, -4.1, 4.1)
        q = -12.5 * zc * zc
        m = 25.0 * zc
        v = q + 32.5
        t1 = jnp.exp(v - m) * cp
        w1 = jnp.exp(v + m) * cp
        bp = jnp.exp(-37.5 - m)
        bn = jnp.exp(m - 37.5)
        t2 = t1 * bp
        t3 = t2 * bp
        w2 = w1 * bn
        w3 = w2 * bn
        e0 = t1 * bn
        us = [jnp.sum(arr, axis=1, keepdims=True) * sc
              for arr, sc in zip((w3, w2, w1, e0, t1, t2, t3), scales)]
        u = jnp.concatenate(us, axis=1)
        ssum = jnp.sum(u, axis=1, keepdims=True)
        hist = u / ssum
        d = jnp.abs(hist - target)
        sl = jnp.where(d < 1.0, 0.5 * d * d, d - 0.5)
        lc = jnp.sum(sl) * jnp.float32(1.0 / (_C * 7))
        active = n_c >= 1000.0
        loss_acc = loss_acc + jnp.where(active, lc, 0.0)
        act_acc = act_acc + jnp.where(active, 1.0, 0.0)

    out_ref[0, 0] = loss_acc / act_acc


def kernel(feature, label):
    x = feature[0].reshape(_C, _P)
    labt = (label[0, 0].astype(jnp.int32)
            .reshape(96, 4, 96, 4).transpose(1, 3, 0, 2).reshape(16, _P))
    cnt = _build_hist_sc()(labt)
    out = pl.pallas_call(
        _loss_kernel,
        out_shape=jax.ShapeDtypeStruct((1, 1), jnp.float32),
        out_specs=pl.BlockSpec(memory_space=pltpu.SMEM),
    )(x, cnt)
    return out[0, 0]


# speedup vs baseline: 1.1370x; 1.0120x over previous
import functools

import jax
import jax.numpy as jnp
from jax import lax
from jax.experimental import pallas as pl
from jax.experimental.pallas import tpu as pltpu
from jax.experimental.pallas import tpu_sc as plsc

_NUM_CLASSES = 19
_P = 96 * 96
_C = 128
_NC = 2
_NS = 16
_NW = _NC * _NS
_CHUNK = 128
_NCHUNK = _P // _CHUNK
_TPW = (_NCHUNK + _NW - 1) // _NW


def _hist_body(labt_hbm, cnt_hbm, lab_v, cnt_v):
    wid = lax.axis_index("s") * _NC + lax.axis_index("c")
    for t in range(_TPW):
        cid = wid + _NW * t

        @pl.when(cid < _NCHUNK)
        def _():
            base = pl.multiple_of(cid * _CHUNK, _CHUNK)
            pltpu.sync_copy(labt_hbm.at[:, pl.ds(base, _CHUNK)], lab_v)

            def group(g, carry):
                off = g * 16
                labs = [lab_v[l, pl.ds(off, 16)] for l in range(16)]
                for c in range(_NUM_CLASSES):
                    acc = jnp.zeros((16,), jnp.float32)
                    for l in range(16):
                        acc = acc + jnp.where(labs[l] == c, 1.0, 0.0)
                    cnt_v[c, pl.ds(off, 16)] = acc
                return carry

            lax.fori_loop(0, _CHUNK // 16, group, 0)
            pltpu.sync_copy(cnt_v, cnt_hbm.at[:, pl.ds(base, _CHUNK)])


@functools.cache
def _build_hist_sc():
    return pl.kernel(
        _hist_body,
        mesh=plsc.VectorSubcoreMesh(core_axis_name="c", subcore_axis_name="s",
                                    num_cores=_NC, num_subcores=_NS),
        out_type=jax.ShapeDtypeStruct((_NUM_CLASSES, _P), jnp.float32),
        scratch_types=[
            pltpu.VMEM((16, _CHUNK), jnp.int32),
            pltpu.VMEM((_NUM_CLASSES, _CHUNK), jnp.float32),
        ],
    )


def _loss_kernel(x_ref, cnt_ref, out_ref):
    x = x_ref[:]
    cnt = cnt_ref[:]

    dn = (((1,), (1,)), ((), ()))
    s12 = jax.lax.dot_general(
        jnp.concatenate([x, x * x], axis=0), cnt, dn,
        precision=jax.lax.Precision.DEFAULT,
        preferred_element_type=jnp.float32)
    s1 = s12[:_C, :]
    s2 = s12[_C:, :]

    kvec = jax.lax.broadcasted_iota(jnp.int32, (1, 7), 1).astype(jnp.float32) - 3.0
    tw = jnp.exp(-0.5 * kvec * kvec)
    target = tw / jnp.sum(tw)

    em45 = jnp.float32(2.8625186e-20)
    em70 = jnp.float32(3.9754497e-31)
    e5 = jnp.float32(148.41316)
    scales = (em70, em45, em45, e5, em45, em45, em70)

    loss_acc = jnp.float32(0.0)
    act_acc = jnp.float32(0.0)
    for c in range(_NUM_CLASSES):
        cp = cnt[c:c + 1, :]
        n_c = jnp.sum(cp)
        nsafe = jnp.maximum(n_c, 1.0)
        mu = s1[:, c:c + 1] / nsafe
        e2 = s2[:, c:c + 1] / nsafe
        var = e2 - mu * mu * (2.0 - n_c / nsafe) + 1e-10
        inv_std = jax.lax.rsqrt(var)
        z = (mu - x) * inv_std
        zc = jnp.clip(z, -4.1, 4.1)
        q = -12.5 * zc * zc
        m = 25.0 * zc
        v = q + 32.5
        t1 = jnp.exp(v - m) * cp
        w1 = jnp.exp(v + m) * cp
        bp = jnp.exp(-37.5 - m)
        bn = jnp.exp(m - 37.5)
        t2 = t1 * bp
        t3 = t2 * bp
        w2 = w1 * bn
        w3 = w2 * bn
        e0 = t1 * bn
        us = [jnp.sum(arr, axis=1, keepdims=True) * sc
              for arr, sc in zip((w3, w2, w1, e0, t1, t2, t3), scales)]
        u = jnp.concatenate(us, axis=1)
        ssum = jnp.sum(u, axis=1, keepdims=True)
        hist = u / ssum
        d = jnp.abs(hist - target)
        sl = jnp.where(d < 1.0, 0.5 * d * d, d - 0.5)
        lc = jnp.sum(sl) * jnp.float32(1.0 / (_C * 7))
        active = n_c >= 1000.0
        loss_acc = loss_acc + jnp.where(active, lc, 0.0)
        act_acc = act_acc + jnp.where(active, 1.0, 0.0)

    out_ref[0, 0] = loss_acc / act_acc


def kernel(feature, label):
    x = feature[0].reshape(_C, _P)
    labt = (label[0, 0].astype(jnp.int32)
            .reshape(96, 4, 96, 4).transpose(1, 3, 0, 2).reshape(16, _P))
    cnt = _build_hist_sc()(labt)
    out = pl.pallas_call(
        _loss_kernel,
        out_shape=jax.ShapeDtypeStruct((1, 1), jnp.float32),
        out_specs=pl.BlockSpec(memory_space=pltpu.SMEM),
    )(x, cnt)
    return out[0, 0]


# speedup vs baseline: 1.1504x; 1.0118x over previous
import functools

import jax
import jax.numpy as jnp
from jax import lax
from jax.experimental import pallas as pl
from jax.experimental.pallas import tpu as pltpu
from jax.experimental.pallas import tpu_sc as plsc

_NUM_CLASSES = 19
_P = 96 * 96
_C = 128
_NC = 2
_NS = 16
_NW = _NC * _NS
_CHUNK = 128
_NCHUNK = _P // _CHUNK
_TPW = (_NCHUNK + _NW - 1) // _NW


def _hist_body(labt_hbm, cnt_hbm, lab_v, cnt_v):
    wid = lax.axis_index("s") * _NC + lax.axis_index("c")
    for t in range(_TPW):
        cid = wid + _NW * t

        @pl.when(cid < _NCHUNK)
        def _():
            base = pl.multiple_of(cid * _CHUNK, _CHUNK)
            pltpu.sync_copy(labt_hbm.at[:, pl.ds(base, _CHUNK)], lab_v)

            def group(g, carry):
                off = g * 16
                labs = [lab_v[l, pl.ds(off, 16)] for l in range(16)]
                for c in range(_NUM_CLASSES):
                    acc = jnp.zeros((16,), jnp.float32)
                    for l in range(16):
                        acc = acc + jnp.where(labs[l] == c, 1.0, 0.0)
                    cnt_v[c, pl.ds(off, 16)] = acc
                return carry

            lax.fori_loop(0, _CHUNK // 16, group, 0)
            pltpu.sync_copy(cnt_v, cnt_hbm.at[:, pl.ds(base, _CHUNK)])


@functools.cache
def _build_hist_sc():
    return pl.kernel(
        _hist_body,
        mesh=plsc.VectorSubcoreMesh(core_axis_name="c", subcore_axis_name="s",
                                    num_cores=_NC, num_subcores=_NS),
        out_type=jax.ShapeDtypeStruct((_NUM_CLASSES, _P), jnp.float32),
        scratch_types=[
            pltpu.VMEM((16, _CHUNK), jnp.int32),
            pltpu.VMEM((_NUM_CLASSES, _CHUNK), jnp.float32),
        ],
    )


def _loss_kernel(x_ref, cnt_ref, out_ref):
    x = x_ref[:]
    cnt = cnt_ref[:]

    dn = (((1,), (1,)), ((), ()))
    s12 = jax.lax.dot_general(
        jnp.concatenate([x, x * x], axis=0), cnt, dn,
        precision=jax.lax.Precision.DEFAULT,
        preferred_element_type=jnp.float32)
    s1 = s12[:_C, :]
    s2 = s12[_C:, :]

    kvec = jax.lax.broadcasted_iota(jnp.int32, (1, 7), 1).astype(jnp.float32) - 3.0
    tw = jnp.exp(-0.5 * kvec * kvec)
    target = tw / jnp.sum(tw)

    em45 = jnp.float32(2.8625186e-20)
    em70 = jnp.float32(3.9754497e-31)
    e5 = jnp.float32(148.41316)
    scales = (em70, em45, em45, e5, em45, em45, em70)

    loss_acc = jnp.float32(0.0)
    act_acc = jnp.float32(0.0)
    for c in range(_NUM_CLASSES):
        cp = cnt[c:c + 1, :]
        n_c = jnp.sum(cp)
        nsafe = jnp.maximum(n_c, 1.0)
        mu = s1[:, c:c + 1] / nsafe
        e2 = s2[:, c:c + 1] / nsafe
        var = e2 - mu * mu * (2.0 - n_c / nsafe) + 1e-10
        inv_std = jax.lax.rsqrt(var)
        z = (mu - x) * inv_std
        zc = jnp.clip(z, -4.1, 4.1)
        q = -12.5 * zc * zc
        m = 25.0 * zc
        e_q = jnp.exp(q + 70.0)
        bp = jnp.exp(-37.5 - m)
        bn = jnp.exp(m - 37.5)
        t1 = e_q * bp * cp
        w1 = e_q * bn * cp
        t2 = t1 * bp
        t3 = t2 * bp
        w2 = w1 * bn
        w3 = w2 * bn
        e0 = t1 * bn
        us = [jnp.sum(arr, axis=1, keepdims=True) * sc
              for arr, sc in zip((w3, w2, w1, e0, t1, t2, t3), scales)]
        u = jnp.concatenate(us, axis=1)
        ssum = jnp.sum(u, axis=1, keepdims=True)
        hist = u / ssum
        d = jnp.abs(hist - target)
        sl = jnp.where(d < 1.0, 0.5 * d * d, d - 0.5)
        lc = jnp.sum(sl) * jnp.float32(1.0 / (_C * 7))
        active = n_c >= 1000.0
        loss_acc = loss_acc + jnp.where(active, lc, 0.0)
        act_acc = act_acc + jnp.where(active, 1.0, 0.0)

    out_ref[0, 0] = loss_acc / act_acc


def kernel(feature, label):
    x = feature[0].reshape(_C, _P)
    labt = (label[0, 0].astype(jnp.int32)
            .reshape(96, 4, 96, 4).transpose(1, 3, 0, 2).reshape(16, _P))
    cnt = _build_hist_sc()(labt)
    out = pl.pallas_call(
        _loss_kernel,
        out_shape=jax.ShapeDtypeStruct((1, 1), jnp.float32),
        out_specs=pl.BlockSpec(memory_space=pltpu.SMEM),
    )(x, cnt)
    return out[0, 0]
